# baseline (speedup 1.0000x reference)
import jax
import jax.numpy as jnp
from jax.experimental import pallas as pl

B = 262144
NUM_TILES = 64
SIG_DIM = 16
RB = 2048


def _body(sig_ref, tsig_ref, scores_ref, idx_ref):
    s = sig_ref[...]
    t = tsig_ref[...]
    scores = jax.lax.dot_general(
        s, t, (((1,), (1,)), ((), ())),
        preferred_element_type=jnp.float32)
    scores_ref[...] = scores
    mx = jnp.max(scores, axis=-1, keepdims=True)
    iota = jax.lax.broadcasted_iota(jnp.int32, scores.shape, 1)
    idx_ref[...] = jnp.min(jnp.where(scores == mx, iota, NUM_TILES), axis=-1)


def kernel(sig, tile_signatures):
    scores, idx = pl.pallas_call(
        _body,
        grid=(B // RB,),
        in_specs=[
            pl.BlockSpec((RB, SIG_DIM), lambda i: (i, 0)),
            pl.BlockSpec((NUM_TILES, SIG_DIM), lambda i: (0, 0)),
        ],
        out_specs=[
            pl.BlockSpec((RB, NUM_TILES), lambda i: (i, 0)),
            pl.BlockSpec((RB,), lambda i: (i,)),
        ],
        out_shape=[
            jax.ShapeDtypeStruct((B, NUM_TILES), jnp.float32),
            jax.ShapeDtypeStruct((B,), jnp.int32),
        ],
    )(sig, tile_signatures)
    return scores, idx


# speedup vs baseline: 1.1893x; 1.1893x over previous
import jax
import jax.numpy as jnp
from jax.experimental import pallas as pl

B = 262144
NUM_TILES = 64
SIG_DIM = 16
RB = 2048


def _body(sig_ref, tsig_ref, scores_ref, idx_ref):
    s = sig_ref[...]
    t = tsig_ref[...]
    scores = jax.lax.dot_general(
        s, t, (((1,), (1,)), ((), ())),
        preferred_element_type=jnp.float32)
    scores_ref[...] = scores
    st = scores.T
    mx = jnp.max(st, axis=0, keepdims=True)
    iota = jax.lax.broadcasted_iota(jnp.int32, st.shape, 0)
    idx_ref[...] = jnp.min(jnp.where(st == mx, iota, NUM_TILES), axis=0)


def kernel(sig, tile_signatures):
    scores, idx = pl.pallas_call(
        _body,
        grid=(B // RB,),
        in_specs=[
            pl.BlockSpec((RB, SIG_DIM), lambda i: (i, 0)),
            pl.BlockSpec((NUM_TILES, SIG_DIM), lambda i: (0, 0)),
        ],
        out_specs=[
            pl.BlockSpec((RB, NUM_TILES), lambda i: (i, 0)),
            pl.BlockSpec((RB,), lambda i: (i,)),
        ],
        out_shape=[
            jax.ShapeDtypeStruct((B, NUM_TILES), jnp.float32),
            jax.ShapeDtypeStruct((B,), jnp.int32),
        ],
    )(sig, tile_signatures)
    return scores, idx


# speedup vs baseline: 1.5120x; 1.2713x over previous
import jax
import jax.numpy as jnp
from jax.experimental import pallas as pl

B = 262144
NUM_TILES = 64
SIG_DIM = 16
RB = 8192


def _body(sig_ref, tsig_ref, scores_ref, idx_ref):
    s = sig_ref[...]
    t = tsig_ref[...]
    scores = jax.lax.dot_general(
        s, t, (((1,), (1,)), ((), ())),
        preferred_element_type=jnp.float32)
    scores_ref[...] = scores
    st = scores.T
    mx = jnp.max(st, axis=0, keepdims=True)
    iota = jax.lax.broadcasted_iota(jnp.int32, st.shape, 0)
    idx_ref[...] = jnp.min(jnp.where(st == mx, iota, NUM_TILES), axis=0)


def kernel(sig, tile_signatures):
    scores, idx = pl.pallas_call(
        _body,
        grid=(B // RB,),
        in_specs=[
            pl.BlockSpec((RB, SIG_DIM), lambda i: (i, 0)),
            pl.BlockSpec((NUM_TILES, SIG_DIM), lambda i: (0, 0)),
        ],
        out_specs=[
            pl.BlockSpec((RB, NUM_TILES), lambda i: (i, 0)),
            pl.BlockSpec((RB,), lambda i: (i,)),
        ],
        out_shape=[
            jax.ShapeDtypeStruct((B, NUM_TILES), jnp.float32),
            jax.ShapeDtypeStruct((B,), jnp.int32),
        ],
    )(sig, tile_signatures)
    return scores, idx
